# Initial kernel scaffold; baseline (speedup 1.0000x reference)
import jax
import jax.numpy as jnp
from jax.experimental import pallas as pl


def kernel(input):
    raise NotImplementedError("write your pallas kernel here")


# speedup vs baseline: 17.9903x; 17.9903x over previous
import functools

import jax
import jax.numpy as jnp
from jax import lax
from jax.experimental import pallas as pl
from jax.experimental.pallas import tpu as pltpu
from jax.experimental.pallas import tpu_sc as plsc

ROWS = 64
COLS = 32768
KTOP = 512
LANES = 16
GROUPS = COLS // LANES
NBKT = 4096
BKT_SHIFT = 20
CAND_CAP = 8192
NCORES = 2
NSUB = 16
NWORK = NCORES * NSUB
ROWS_PER_W = ROWS // NWORK
I32MIN = jnp.iinfo(jnp.int32).min


def _worker_id():
  return lax.axis_index("s") * NCORES + lax.axis_index("c")


def _key(b):
  return b ^ ((b >> 31) & jnp.int32(0x7FFFFFFF))


def _topk_mask_bits(x_bits):
  mesh = plsc.VectorSubcoreMesh(
      core_axis_name="c", subcore_axis_name="s",
      num_cores=NCORES, num_subcores=NSUB)

  @functools.partial(
      pl.kernel,
      out_type=jax.ShapeDtypeStruct((ROWS, COLS), jnp.int32),
      mesh=mesh,
      compiler_params=pltpu.CompilerParams(needs_layout_passes=False),
      scratch_types=[
          pltpu.VMEM((COLS,), jnp.int32),
          pltpu.VMEM((LANES * NBKT,), jnp.int32),
          pltpu.VMEM((CAND_CAP,), jnp.int32),
      ],
  )
  def kern(x_hbm, out_hbm, row_v, hist_v, cand_v):
    wid = _worker_id()
    iota = lax.iota(jnp.int32, LANES)
    ones = jnp.ones((LANES,), jnp.int32)
    zeros = jnp.zeros((LANES,), jnp.int32)
    lane_base = iota * NBKT

    for r_local in range(ROWS_PER_W):
      row = wid * ROWS_PER_W + r_local
      pltpu.sync_copy(x_hbm.at[row], row_v)

      @plsc.parallel_loop(0, LANES * NBKT, step=LANES, unroll=8)
      def _(i):
        hist_v[pl.ds(i, LANES)] = zeros

      @plsc.parallel_loop(0, COLS, step=LANES, unroll=4)
      def _(i):
        b = row_v[pl.ds(i, LANES)]
        ks = _key(b)
        row_v[pl.ds(i, LANES)] = ks
        bkt = (ks >> BKT_SHIFT) + jnp.int32(NBKT // 2)
        plsc.addupdate_scatter(hist_v, [bkt + lane_base], ones)

      def scan_body(j, carry):
        above, bkt_found, above_b, found = carry
        g = jnp.int32(NBKT // LANES - 1) - j
        totals = zeros
        for l in range(LANES):
          totals = totals + hist_v[pl.ds(l * NBKT + g * LANES, LANES)]
        rev = lax.rev(totals, (0,))
        cs = plsc.cumsum(rev)
        cum = above + cs
        below = jnp.where(cum < KTOP, 1, 0)
        lstar = jnp.sum(below)
        hit = (~found) & (lstar < LANES)
        excl = jnp.sum(jnp.where(cum < KTOP, rev, 0))
        bkt_here = g * LANES + (jnp.int32(LANES - 1) - lstar)
        bkt_found = jnp.where(hit, bkt_here, bkt_found)
        above_b = jnp.where(hit, above + excl, above_b)
        found = found | (lstar < LANES)
        above = above + jnp.sum(totals)
        return above, bkt_found, above_b, found

      _, bkt, above_b, _ = lax.fori_loop(
          0, NBKT // LANES, scan_body,
          (jnp.int32(0), jnp.int32(0), jnp.int32(0), False))

      kprime = jnp.int32(KTOP) - above_b
      bkt_rel = bkt - jnp.int32(NBKT // 2)
      bmin = bkt_rel << BKT_SHIFT

      @plsc.parallel_loop(0, COLS, step=LANES, unroll=2, carry=jnp.int32(0))
      def cnt(i, c):
        ks = row_v[pl.ds(i, LANES)]
        m = (ks >> BKT_SHIFT) == bkt_rel
        plsc.store_compressed(cand_v.at[pl.ds(c, LANES)], ks, mask=m)
        return c + jnp.sum(jnp.where(m, 1, 0))

      cand_v[pl.ds(cnt, LANES)] = jnp.full((LANES,), I32MIN, jnp.int32)
      ngrp = (cnt + LANES - 1) // LANES

      prefix = bmin
      for bit in range(BKT_SHIFT - 1, -1, -1):
        mid = prefix | (jnp.int32(1) << bit)

        def count_body(i, acc, mid=mid):
          kv = cand_v[pl.ds(i * LANES, LANES)]
          return acc + jnp.where(kv >= mid, 1, 0)

        cge = jnp.sum(lax.fori_loop(0, ngrp, count_body, zeros))
        prefix = jnp.where(cge >= kprime, mid, prefix)
      thr = prefix

      def count2_body(i, acc):
        kv = cand_v[pl.ds(i * LANES, LANES)]
        return (acc[0] + jnp.where(kv > thr, 1, 0),
                acc[1] + jnp.where(kv == thr, 1, 0))

      a_gt, a_eq = lax.fori_loop(0, ngrp, count2_body, (zeros, zeros))
      c_gt = jnp.sum(a_gt)
      c_eq = jnp.sum(a_eq)
      t_rem = kprime - c_gt

      def find_cut(_):
        def fb(i, carry):
          cnt_eq, cut, done = carry
          ks = row_v[pl.ds(i * LANES, LANES)]
          eq = ks == thr
          eqi = jnp.where(eq, 1, 0)
          cs = plsc.cumsum(eqi)
          tgt = t_rem - cnt_eq
          sel = eq & (cs == tgt)
          gcount = jnp.sum(eqi)
          here = (~done) & (gcount >= tgt)
          idx_in = jnp.sum(jnp.where(sel, iota, 0))
          cut = jnp.where(here, i * LANES + idx_in, cut)
          return cnt_eq + gcount, cut, done | here

        _, cut, _ = lax.fori_loop(
            0, GROUPS, fb, (jnp.int32(0), jnp.int32(COLS - 1), False))
        return cut

      cut = lax.cond(c_eq > t_rem, find_cut,
                     lambda _: jnp.int32(COLS - 1), None)

      @plsc.parallel_loop(0, COLS, step=LANES, unroll=4)
      def _(i):
        ks = row_v[pl.ds(i, LANES)]
        idxv = iota + i
        keep = (ks > thr) | ((ks == thr) & (idxv <= cut))
        row_v[pl.ds(i, LANES)] = jnp.where(keep, _key(ks), jnp.int32(0))

      pltpu.sync_copy(row_v, out_hbm.at[row])

  return kern(x_bits)


def kernel(input):
  bits = lax.bitcast_convert_type(input, jnp.int32)
  return lax.bitcast_convert_type(_topk_mask_bits(bits), jnp.float32)


# speedup vs baseline: 19.4428x; 1.0807x over previous
import functools

import jax
import jax.numpy as jnp
from jax import lax
from jax.experimental import pallas as pl
from jax.experimental.pallas import tpu as pltpu
from jax.experimental.pallas import tpu_sc as plsc

ROWS = 64
COLS = 32768
KTOP = 512
LANES = 16
GROUPS = COLS // LANES
NBKT = 4096
BKT_SHIFT = 20
CAND_CAP = 8192
NCORES = 2
NSUB = 16
NWORK = NCORES * NSUB
ROWS_PER_W = ROWS // NWORK
I32MIN = jnp.iinfo(jnp.int32).min


def _worker_id():
  return lax.axis_index("s") * NCORES + lax.axis_index("c")


def _key(b):
  return b ^ ((b >> 31) & jnp.int32(0x7FFFFFFF))


def _topk_mask_bits(x_bits):
  mesh = plsc.VectorSubcoreMesh(
      core_axis_name="c", subcore_axis_name="s",
      num_cores=NCORES, num_subcores=NSUB)

  @functools.partial(
      pl.kernel,
      out_type=jax.ShapeDtypeStruct((ROWS, COLS), jnp.int32),
      mesh=mesh,
      compiler_params=pltpu.CompilerParams(needs_layout_passes=False),
      scratch_types=[
          pltpu.VMEM((COLS,), jnp.int32),
          pltpu.VMEM((NBKT,), jnp.int32),
          pltpu.VMEM((CAND_CAP,), jnp.int32),
      ],
  )
  def kern(x_hbm, out_hbm, row_v, hist_v, cand_v):
    wid = _worker_id()
    iota = lax.iota(jnp.int32, LANES)
    ones = jnp.ones((LANES,), jnp.int32)
    zeros = jnp.zeros((LANES,), jnp.int32)
    for r_local in range(ROWS_PER_W):
      row = wid * ROWS_PER_W + r_local
      pltpu.sync_copy(x_hbm.at[row], row_v)

      @plsc.parallel_loop(0, NBKT, step=LANES, unroll=8)
      def _(i):
        hist_v[pl.ds(i, LANES)] = zeros

      @plsc.parallel_loop(0, COLS, step=LANES, unroll=4)
      def _(i):
        b = row_v[pl.ds(i, LANES)]
        ks = _key(b)
        row_v[pl.ds(i, LANES)] = ks
        bkt = (ks >> BKT_SHIFT) + jnp.int32(NBKT // 2)
        plsc.addupdate_scatter(hist_v, [bkt], ones)

      def scan_body(j, carry):
        above, bkt_found, above_b, found = carry
        g = jnp.int32(NBKT // LANES - 1) - j
        totals = hist_v[pl.ds(g * LANES, LANES)]
        rev = lax.rev(totals, (0,))
        cs = plsc.cumsum(rev)
        cum = above + cs
        below = jnp.where(cum < KTOP, 1, 0)
        lstar = jnp.sum(below)
        hit = (~found) & (lstar < LANES)
        excl = jnp.sum(jnp.where(cum < KTOP, rev, 0))
        bkt_here = g * LANES + (jnp.int32(LANES - 1) - lstar)
        bkt_found = jnp.where(hit, bkt_here, bkt_found)
        above_b = jnp.where(hit, above + excl, above_b)
        found = found | (lstar < LANES)
        above = above + jnp.sum(totals)
        return above, bkt_found, above_b, found

      _, bkt, above_b, _ = lax.fori_loop(
          0, NBKT // LANES, scan_body,
          (jnp.int32(0), jnp.int32(0), jnp.int32(0), False))

      kprime = jnp.int32(KTOP) - above_b
      bkt_rel = bkt - jnp.int32(NBKT // 2)
      bmin = bkt_rel << BKT_SHIFT

      @plsc.parallel_loop(0, COLS, step=LANES, unroll=2, carry=jnp.int32(0))
      def cnt(i, c):
        ks = row_v[pl.ds(i, LANES)]
        m = (ks >> BKT_SHIFT) == bkt_rel
        plsc.store_compressed(cand_v.at[pl.ds(c, LANES)], ks, mask=m)
        return c + jnp.sum(jnp.where(m, 1, 0))

      cand_v[pl.ds(cnt, LANES)] = jnp.full((LANES,), I32MIN, jnp.int32)
      ngrp = (cnt + LANES - 1) // LANES

      prefix = bmin
      for bit in range(BKT_SHIFT - 1, -1, -1):
        mid = prefix | (jnp.int32(1) << bit)

        def count_body(i, acc, mid=mid):
          kv = cand_v[pl.ds(i * LANES, LANES)]
          return acc + jnp.where(kv >= mid, 1, 0)

        cge = jnp.sum(lax.fori_loop(0, ngrp, count_body, zeros))
        prefix = jnp.where(cge >= kprime, mid, prefix)
      thr = prefix

      def count2_body(i, acc):
        kv = cand_v[pl.ds(i * LANES, LANES)]
        return (acc[0] + jnp.where(kv > thr, 1, 0),
                acc[1] + jnp.where(kv == thr, 1, 0))

      a_gt, a_eq = lax.fori_loop(0, ngrp, count2_body, (zeros, zeros))
      c_gt = jnp.sum(a_gt)
      c_eq = jnp.sum(a_eq)
      t_rem = kprime - c_gt

      def find_cut(_):
        def fb(i, carry):
          cnt_eq, cut, done = carry
          ks = row_v[pl.ds(i * LANES, LANES)]
          eq = ks == thr
          eqi = jnp.where(eq, 1, 0)
          cs = plsc.cumsum(eqi)
          tgt = t_rem - cnt_eq
          sel = eq & (cs == tgt)
          gcount = jnp.sum(eqi)
          here = (~done) & (gcount >= tgt)
          idx_in = jnp.sum(jnp.where(sel, iota, 0))
          cut = jnp.where(here, i * LANES + idx_in, cut)
          return cnt_eq + gcount, cut, done | here

        _, cut, _ = lax.fori_loop(
            0, GROUPS, fb, (jnp.int32(0), jnp.int32(COLS - 1), False))
        return cut

      cut = lax.cond(c_eq > t_rem, find_cut,
                     lambda _: jnp.int32(COLS - 1), None)

      @plsc.parallel_loop(0, COLS, step=LANES, unroll=4)
      def _(i):
        ks = row_v[pl.ds(i, LANES)]
        idxv = iota + i
        keep = (ks > thr) | ((ks == thr) & (idxv <= cut))
        row_v[pl.ds(i, LANES)] = jnp.where(keep, _key(ks), jnp.int32(0))

      pltpu.sync_copy(row_v, out_hbm.at[row])

  return kern(x_bits)


def kernel(input):
  bits = lax.bitcast_convert_type(input, jnp.int32)
  return lax.bitcast_convert_type(_topk_mask_bits(bits), jnp.float32)


# speedup vs baseline: 20.3862x; 1.0485x over previous
import functools

import jax
import jax.numpy as jnp
from jax import lax
from jax.experimental import pallas as pl
from jax.experimental.pallas import tpu as pltpu
from jax.experimental.pallas import tpu_sc as plsc

ROWS = 64
COLS = 32768
KTOP = 512
LANES = 16
GROUPS = COLS // LANES
NBKT = 4096
BKT_SHIFT = 20
CAND_CAP = 8192
NCORES = 2
NSUB = 16
NWORK = NCORES * NSUB
ROWS_PER_W = ROWS // NWORK
I32MIN = jnp.iinfo(jnp.int32).min


def _worker_id():
  return lax.axis_index("s") * NCORES + lax.axis_index("c")


def _key(b):
  return b ^ ((b >> 31) & jnp.int32(0x7FFFFFFF))


def _topk_mask_bits(x_bits):
  mesh = plsc.VectorSubcoreMesh(
      core_axis_name="c", subcore_axis_name="s",
      num_cores=NCORES, num_subcores=NSUB)

  @functools.partial(
      pl.kernel,
      out_type=jax.ShapeDtypeStruct((ROWS, COLS), jnp.int32),
      mesh=mesh,
      compiler_params=pltpu.CompilerParams(needs_layout_passes=False),
      scratch_types=[
          pltpu.VMEM((COLS,), jnp.int32),
          pltpu.VMEM((NBKT,), jnp.int32),
          pltpu.VMEM((CAND_CAP,), jnp.int32),
      ],
  )
  def kern(x_hbm, out_hbm, row_v, hist_v, cand_v):
    wid = _worker_id()
    iota = lax.iota(jnp.int32, LANES)
    ones = jnp.ones((LANES,), jnp.int32)
    zeros = jnp.zeros((LANES,), jnp.int32)
    for r_local in range(ROWS_PER_W):
      row = wid * ROWS_PER_W + r_local
      pltpu.sync_copy(x_hbm.at[row], row_v)

      @plsc.parallel_loop(0, NBKT, step=LANES, unroll=8)
      def _(i):
        hist_v[pl.ds(i, LANES)] = zeros

      @plsc.parallel_loop(0, COLS, step=LANES, unroll=8)
      def _(i):
        b = row_v[pl.ds(i, LANES)]
        ks = _key(b)
        row_v[pl.ds(i, LANES)] = ks
        bkt = (ks >> BKT_SHIFT) + jnp.int32(NBKT // 2)
        plsc.addupdate_scatter(hist_v, [bkt], ones)

      top_g = jnp.int32(NBKT // LANES - 1)
      s_top = jnp.sum(hist_v[pl.ds(top_g * LANES, LANES)])

      def scan_cond(c):
        _, above, s = c
        return above + s < KTOP

      def scan_step(c):
        g, above, s = c
        s_next = jnp.sum(hist_v[pl.ds((g - 1) * LANES, LANES)])
        return g - 1, above + s, s_next

      g_star, above_g, _ = lax.while_loop(
          scan_cond, scan_step, (top_g, jnp.int32(0), s_top))
      totals = hist_v[pl.ds(g_star * LANES, LANES)]
      rev = lax.rev(totals, (0,))
      cum = above_g + plsc.cumsum(rev)
      below = jnp.where(cum < KTOP, 1, 0)
      lstar = jnp.sum(below)
      excl = jnp.sum(jnp.where(cum < KTOP, rev, 0))
      bkt = g_star * LANES + (jnp.int32(LANES - 1) - lstar)
      above_b = above_g + excl

      kprime = jnp.int32(KTOP) - above_b
      bkt_rel = bkt - jnp.int32(NBKT // 2)
      bmin = bkt_rel << BKT_SHIFT

      @plsc.parallel_loop(0, COLS, step=LANES, unroll=4, carry=jnp.int32(0))
      def cnt(i, c):
        ks = row_v[pl.ds(i, LANES)]
        m = (ks >> BKT_SHIFT) == bkt_rel
        plsc.store_compressed(cand_v.at[pl.ds(c, LANES)], ks, mask=m)
        return c + jnp.sum(jnp.where(m, 1, 0))

      cand_v[pl.ds(cnt, LANES)] = jnp.full((LANES,), I32MIN, jnp.int32)
      ngrp = (cnt + LANES - 1) // LANES

      prefix = bmin
      for bit in range(BKT_SHIFT - 1, -1, -1):
        mid = prefix | (jnp.int32(1) << bit)

        def count_body(i, acc, mid=mid):
          kv = cand_v[pl.ds(i * LANES, LANES)]
          return acc + jnp.where(kv >= mid, 1, 0)

        cge = jnp.sum(lax.fori_loop(0, ngrp, count_body, zeros))
        prefix = jnp.where(cge >= kprime, mid, prefix)
      thr = prefix

      def count2_body(i, acc):
        kv = cand_v[pl.ds(i * LANES, LANES)]
        return (acc[0] + jnp.where(kv > thr, 1, 0),
                acc[1] + jnp.where(kv == thr, 1, 0))

      a_gt, a_eq = lax.fori_loop(0, ngrp, count2_body, (zeros, zeros))
      c_gt = jnp.sum(a_gt)
      c_eq = jnp.sum(a_eq)
      t_rem = kprime - c_gt

      def find_cut(_):
        def fb(i, carry):
          cnt_eq, cut, done = carry
          ks = row_v[pl.ds(i * LANES, LANES)]
          eq = ks == thr
          eqi = jnp.where(eq, 1, 0)
          cs = plsc.cumsum(eqi)
          tgt = t_rem - cnt_eq
          sel = eq & (cs == tgt)
          gcount = jnp.sum(eqi)
          here = (~done) & (gcount >= tgt)
          idx_in = jnp.sum(jnp.where(sel, iota, 0))
          cut = jnp.where(here, i * LANES + idx_in, cut)
          return cnt_eq + gcount, cut, done | here

        _, cut, _ = lax.fori_loop(
            0, GROUPS, fb, (jnp.int32(0), jnp.int32(COLS - 1), False))
        return cut

      cut = lax.cond(c_eq > t_rem, find_cut,
                     lambda _: jnp.int32(COLS - 1), None)

      @plsc.parallel_loop(0, COLS, step=LANES, unroll=8)
      def _(i):
        ks = row_v[pl.ds(i, LANES)]
        idxv = iota + i
        keep = (ks > thr) | ((ks == thr) & (idxv <= cut))
        row_v[pl.ds(i, LANES)] = jnp.where(keep, _key(ks), jnp.int32(0))

      pltpu.sync_copy(row_v, out_hbm.at[row])

  return kern(x_bits)


def kernel(input):
  bits = lax.bitcast_convert_type(input, jnp.int32)
  return lax.bitcast_convert_type(_topk_mask_bits(bits), jnp.float32)


# speedup vs baseline: 21.2409x; 1.0419x over previous
import functools

import jax
import jax.numpy as jnp
from jax import lax
from jax.experimental import pallas as pl
from jax.experimental.pallas import tpu as pltpu
from jax.experimental.pallas import tpu_sc as plsc

ROWS = 64
COLS = 32768
KTOP = 512
LANES = 16
GROUPS = COLS // LANES
NBKT = 4096
BKT_SHIFT = 20
CAND_CAP = 8192
NCORES = 2
NSUB = 16
NWORK = NCORES * NSUB
ROWS_PER_W = ROWS // NWORK
I32MIN = jnp.iinfo(jnp.int32).min


def _worker_id():
  return lax.axis_index("s") * NCORES + lax.axis_index("c")


def _key(b):
  return b ^ ((b >> 31) & jnp.int32(0x7FFFFFFF))


def _topk_mask_bits(x_bits):
  mesh = plsc.VectorSubcoreMesh(
      core_axis_name="c", subcore_axis_name="s",
      num_cores=NCORES, num_subcores=NSUB)

  @functools.partial(
      pl.kernel,
      out_type=jax.ShapeDtypeStruct((ROWS, COLS), jnp.int32),
      mesh=mesh,
      compiler_params=pltpu.CompilerParams(needs_layout_passes=False),
      scratch_types=[
          pltpu.VMEM((ROWS_PER_W * COLS,), jnp.int32),
          pltpu.VMEM((NBKT,), jnp.int32),
          pltpu.VMEM((CAND_CAP,), jnp.int32),
          pltpu.SemaphoreType.DMA,
          pltpu.SemaphoreType.DMA,
          pltpu.SemaphoreType.DMA,
          pltpu.SemaphoreType.DMA,
      ],
  )
  def kern(x_hbm, out_hbm, buf_v, hist_v, cand_v, s_in0, s_in1, s_o0, s_o1):
    wid = _worker_id()
    iota = lax.iota(jnp.int32, LANES)
    ones = jnp.ones((LANES,), jnp.int32)
    zeros = jnp.zeros((LANES,), jnp.int32)
    pair = wid * ROWS_PER_W

    cp_in = [
        pltpu.async_copy(x_hbm.at[pair + r],
                         buf_v.at[pl.ds(r * COLS, COLS)],
                         (s_in0, s_in1)[r])
        for r in range(ROWS_PER_W)
    ]
    cp_out = []

    for r_local in range(ROWS_PER_W):
      base = r_local * COLS
      cp_in[r_local].wait()

      @plsc.parallel_loop(0, NBKT, step=LANES, unroll=8)
      def _(i):
        hist_v[pl.ds(i, LANES)] = zeros

      @plsc.parallel_loop(base, base + COLS, step=LANES, unroll=8)
      def _(i):
        ks = _key(buf_v[pl.ds(i, LANES)])
        bkt = (ks >> BKT_SHIFT) + jnp.int32(NBKT // 2)
        plsc.addupdate_scatter(hist_v, [bkt], ones)

      top_g = jnp.int32(NBKT // LANES - 1)
      s_top = jnp.sum(hist_v[pl.ds(top_g * LANES, LANES)])

      def scan_cond(c):
        _, above, s = c
        return above + s < KTOP

      def scan_step(c):
        g, above, s = c
        s_next = jnp.sum(hist_v[pl.ds((g - 1) * LANES, LANES)])
        return g - 1, above + s, s_next

      g_star, above_g, _ = lax.while_loop(
          scan_cond, scan_step, (top_g, jnp.int32(0), s_top))
      totals = hist_v[pl.ds(g_star * LANES, LANES)]
      rev = lax.rev(totals, (0,))
      cum = above_g + plsc.cumsum(rev)
      below = jnp.where(cum < KTOP, 1, 0)
      lstar = jnp.sum(below)
      excl = jnp.sum(jnp.where(cum < KTOP, rev, 0))
      bkt = g_star * LANES + (jnp.int32(LANES - 1) - lstar)
      above_b = above_g + excl

      kprime = jnp.int32(KTOP) - above_b
      bkt_rel = bkt - jnp.int32(NBKT // 2)
      bmin = bkt_rel << BKT_SHIFT

      @plsc.parallel_loop(base, base + COLS, step=LANES, unroll=4,
                          carry=jnp.int32(0))
      def cnt(i, c):
        ks = _key(buf_v[pl.ds(i, LANES)])
        m = (ks >> BKT_SHIFT) == bkt_rel
        plsc.store_compressed(cand_v.at[pl.ds(c, LANES)], ks, mask=m)
        return c + jnp.sum(jnp.where(m, 1, 0))

      cand_v[pl.ds(cnt, LANES)] = jnp.full((LANES,), I32MIN, jnp.int32)
      ngrp = (cnt + LANES - 1) // LANES

      prefix = bmin
      for bit in range(BKT_SHIFT - 1, -1, -1):
        mid = prefix | (jnp.int32(1) << bit)

        def count_body(i, acc, mid=mid):
          kv = cand_v[pl.ds(i * LANES, LANES)]
          return acc + jnp.where(kv >= mid, 1, 0)

        cge = jnp.sum(lax.fori_loop(0, ngrp, count_body, zeros))
        prefix = jnp.where(cge >= kprime, mid, prefix)
      thr = prefix

      def count2_body(i, acc):
        kv = cand_v[pl.ds(i * LANES, LANES)]
        return (acc[0] + jnp.where(kv > thr, 1, 0),
                acc[1] + jnp.where(kv == thr, 1, 0))

      a_gt, a_eq = lax.fori_loop(0, ngrp, count2_body, (zeros, zeros))
      c_gt = jnp.sum(a_gt)
      c_eq = jnp.sum(a_eq)
      t_rem = kprime - c_gt

      def find_cut(_):
        def fb(i, carry):
          cnt_eq, cut, done = carry
          ks = _key(buf_v[pl.ds(base + i * LANES, LANES)])
          eq = ks == thr
          eqi = jnp.where(eq, 1, 0)
          cs = plsc.cumsum(eqi)
          tgt = t_rem - cnt_eq
          sel = eq & (cs == tgt)
          gcount = jnp.sum(eqi)
          here = (~done) & (gcount >= tgt)
          idx_in = jnp.sum(jnp.where(sel, iota, 0))
          cut = jnp.where(here, i * LANES + idx_in, cut)
          return cnt_eq + gcount, cut, done | here

        _, cut, _ = lax.fori_loop(
            0, GROUPS, fb, (jnp.int32(0), jnp.int32(COLS - 1), False))
        return cut

      cut = lax.cond(c_eq > t_rem, find_cut,
                     lambda _: jnp.int32(COLS - 1), None)

      @plsc.parallel_loop(base, base + COLS, step=LANES, unroll=8)
      def _(i):
        b = buf_v[pl.ds(i, LANES)]
        ks = _key(b)
        idxv = iota + (i - base)
        keep = (ks > thr) | ((ks == thr) & (idxv <= cut))
        buf_v[pl.ds(i, LANES)] = jnp.where(keep, b, jnp.int32(0))

      cp_out.append(
          pltpu.async_copy(buf_v.at[pl.ds(base, COLS)],
                           out_hbm.at[pair + r_local],
                           (s_o0, s_o1)[r_local]))

    for cp in cp_out:
      cp.wait()

  return kern(x_bits)


def kernel(input):
  bits = lax.bitcast_convert_type(input, jnp.int32)
  return lax.bitcast_convert_type(_topk_mask_bits(bits), jnp.float32)


# speedup vs baseline: 23.8419x; 1.1225x over previous
import functools

import jax
import jax.numpy as jnp
from jax import lax
from jax.experimental import pallas as pl
from jax.experimental.pallas import tpu as pltpu
from jax.experimental.pallas import tpu_sc as plsc

ROWS = 64
COLS = 32768
KTOP = 512
LANES = 16
GROUPS = COLS // LANES
NBKT = 4096
BKT_SHIFT = 20
CAND_CAP = 8192
CAND_FIX = 512
NCORES = 2
NSUB = 16
NWORK = NCORES * NSUB
ROWS_PER_W = ROWS // NWORK
I32MIN = jnp.iinfo(jnp.int32).min


def _worker_id():
  return lax.axis_index("s") * NCORES + lax.axis_index("c")


def _key(b):
  return b ^ ((b >> 31) & jnp.int32(0x7FFFFFFF))


def _topk_mask_bits(x_bits):
  mesh = plsc.VectorSubcoreMesh(
      core_axis_name="c", subcore_axis_name="s",
      num_cores=NCORES, num_subcores=NSUB)

  @functools.partial(
      pl.kernel,
      out_type=jax.ShapeDtypeStruct((ROWS, COLS), jnp.int32),
      mesh=mesh,
      compiler_params=pltpu.CompilerParams(needs_layout_passes=False),
      scratch_types=[
          pltpu.VMEM((ROWS_PER_W * COLS,), jnp.int32),
          pltpu.VMEM((NBKT,), jnp.int32),
          pltpu.VMEM((CAND_CAP,), jnp.int32),
          pltpu.SemaphoreType.DMA,
          pltpu.SemaphoreType.DMA,
          pltpu.SemaphoreType.DMA,
          pltpu.SemaphoreType.DMA,
      ],
  )
  def kern(x_hbm, out_hbm, buf_v, hist_v, cand_v, s_in0, s_in1, s_o0, s_o1):
    wid = _worker_id()
    iota = lax.iota(jnp.int32, LANES)
    ones = jnp.ones((LANES,), jnp.int32)
    zeros = jnp.zeros((LANES,), jnp.int32)
    pair = wid * ROWS_PER_W

    cp_in = [
        pltpu.async_copy(x_hbm.at[pair + r],
                         buf_v.at[pl.ds(r * COLS, COLS)],
                         (s_in0, s_in1)[r])
        for r in range(ROWS_PER_W)
    ]
    cp_out = []

    for r_local in range(ROWS_PER_W):
      base = r_local * COLS
      cp_in[r_local].wait()

      @plsc.parallel_loop(0, NBKT, step=LANES, unroll=8)
      def _(i):
        hist_v[pl.ds(i, LANES)] = zeros

      @plsc.parallel_loop(base, base + COLS, step=LANES, unroll=8)
      def _(i):
        ks = _key(buf_v[pl.ds(i, LANES)])
        bkt = (ks >> BKT_SHIFT) + jnp.int32(NBKT // 2)
        plsc.addupdate_scatter(hist_v, [bkt], ones)

      top_g = jnp.int32(NBKT // LANES - 1)
      s_top = jnp.sum(hist_v[pl.ds(top_g * LANES, LANES)])

      def scan_cond(c):
        _, above, s = c
        return above + s < KTOP

      def scan_step(c):
        g, above, s = c
        s_next = jnp.sum(hist_v[pl.ds((g - 1) * LANES, LANES)])
        return g - 1, above + s, s_next

      g_star, above_g, _ = lax.while_loop(
          scan_cond, scan_step, (top_g, jnp.int32(0), s_top))
      totals = hist_v[pl.ds(g_star * LANES, LANES)]
      rev = lax.rev(totals, (0,))
      cum = above_g + plsc.cumsum(rev)
      below = jnp.where(cum < KTOP, 1, 0)
      lstar = jnp.sum(below)
      excl = jnp.sum(jnp.where(cum < KTOP, rev, 0))
      bkt = g_star * LANES + (jnp.int32(LANES - 1) - lstar)
      above_b = above_g + excl

      kprime = jnp.int32(KTOP) - above_b
      bkt_rel = bkt - jnp.int32(NBKT // 2)
      bmin = bkt_rel << BKT_SHIFT

      sentinel = jnp.full((LANES,), I32MIN, jnp.int32)

      @plsc.parallel_loop(0, CAND_FIX, step=LANES, unroll=8)
      def _(i):
        cand_v[pl.ds(i, LANES)] = sentinel

      @plsc.parallel_loop(base, base + COLS, step=LANES, unroll=4,
                          carry=jnp.int32(0))
      def cnt(i, c):
        ks = _key(buf_v[pl.ds(i, LANES)])
        m = (ks >> BKT_SHIFT) == bkt_rel
        plsc.store_compressed(cand_v.at[pl.ds(c, LANES)], ks, mask=m)
        return c + jnp.sum(jnp.where(m, 1, 0))

      cand_v[pl.ds(cnt, LANES)] = sentinel

      def search_static(_):
        prefix = bmin
        for bit in range(BKT_SHIFT - 1, -1, -1):
          mid = prefix | (jnp.int32(1) << bit)

          @plsc.parallel_loop(0, CAND_FIX, step=LANES, unroll=8, carry=zeros)
          def acc(i, a, mid=mid):
            kv = cand_v[pl.ds(i, LANES)]
            return a + jnp.where(kv >= mid, 1, 0)

          prefix = jnp.where(jnp.sum(acc) >= kprime, mid, prefix)
        thr = prefix

        @plsc.parallel_loop(0, CAND_FIX, step=LANES, unroll=8,
                            carry=(zeros, zeros))
        def acc2(i, a):
          kv = cand_v[pl.ds(i, LANES)]
          return (a[0] + jnp.where(kv > thr, 1, 0),
                  a[1] + jnp.where(kv == thr, 1, 0))

        return thr, jnp.sum(acc2[0]), jnp.sum(acc2[1])

      def search_dynamic(_):
        ngrp = (cnt + LANES - 1) // LANES
        prefix = bmin
        for bit in range(BKT_SHIFT - 1, -1, -1):
          mid = prefix | (jnp.int32(1) << bit)

          def count_body(i, acc, mid=mid):
            kv = cand_v[pl.ds(i * LANES, LANES)]
            return acc + jnp.where(kv >= mid, 1, 0)

          cge = jnp.sum(lax.fori_loop(0, ngrp, count_body, zeros))
          prefix = jnp.where(cge >= kprime, mid, prefix)
        thr = prefix

        def count2_body(i, acc):
          kv = cand_v[pl.ds(i * LANES, LANES)]
          return (acc[0] + jnp.where(kv > thr, 1, 0),
                  acc[1] + jnp.where(kv == thr, 1, 0))

        a_gt, a_eq = lax.fori_loop(0, ngrp, count2_body, (zeros, zeros))
        return thr, jnp.sum(a_gt), jnp.sum(a_eq)

      thr, c_gt, c_eq = lax.cond(
          cnt <= CAND_FIX - LANES, search_static, search_dynamic, None)
      t_rem = kprime - c_gt

      def find_cut(_):
        def fb(i, carry):
          cnt_eq, cut, done = carry
          ks = _key(buf_v[pl.ds(base + i * LANES, LANES)])
          eq = ks == thr
          eqi = jnp.where(eq, 1, 0)
          cs = plsc.cumsum(eqi)
          tgt = t_rem - cnt_eq
          sel = eq & (cs == tgt)
          gcount = jnp.sum(eqi)
          here = (~done) & (gcount >= tgt)
          idx_in = jnp.sum(jnp.where(sel, iota, 0))
          cut = jnp.where(here, i * LANES + idx_in, cut)
          return cnt_eq + gcount, cut, done | here

        _, cut, _ = lax.fori_loop(
            0, GROUPS, fb, (jnp.int32(0), jnp.int32(COLS - 1), False))
        return cut

      cut = lax.cond(c_eq > t_rem, find_cut,
                     lambda _: jnp.int32(COLS - 1), None)

      fast_ok = (thr > 0) & (cut == jnp.int32(COLS - 1))

      @pl.when(fast_ok)
      def _():
        @plsc.parallel_loop(base, base + COLS, step=LANES, unroll=8)
        def _(i):
          b = buf_v[pl.ds(i, LANES)]
          buf_v[pl.ds(i, LANES)] = jnp.where(b >= thr, b, jnp.int32(0))

      @pl.when(jnp.logical_not(fast_ok))
      def _():
        @plsc.parallel_loop(base, base + COLS, step=LANES, unroll=8)
        def _(i):
          b = buf_v[pl.ds(i, LANES)]
          ks = _key(b)
          idxv = iota + (i - base)
          keep = (ks > thr) | ((ks == thr) & (idxv <= cut))
          buf_v[pl.ds(i, LANES)] = jnp.where(keep, b, jnp.int32(0))

      cp_out.append(
          pltpu.async_copy(buf_v.at[pl.ds(base, COLS)],
                           out_hbm.at[pair + r_local],
                           (s_o0, s_o1)[r_local]))

    for cp in cp_out:
      cp.wait()

  return kern(x_bits)


def kernel(input):
  bits = lax.bitcast_convert_type(input, jnp.int32)
  return lax.bitcast_convert_type(_topk_mask_bits(bits), jnp.float32)


# speedup vs baseline: 24.2362x; 1.0165x over previous
import functools

import jax
import jax.numpy as jnp
from jax import lax
from jax.experimental import pallas as pl
from jax.experimental.pallas import tpu as pltpu
from jax.experimental.pallas import tpu_sc as plsc

ROWS = 64
COLS = 32768
KTOP = 512
LANES = 16
GROUPS = COLS // LANES
NBKT = 4096
BKT_SHIFT = 20
CAND_CAP = 8192
CAND_FIX = 512
NCORES = 2
NSUB = 16
NWORK = NCORES * NSUB
ROWS_PER_W = ROWS // NWORK
I32MIN = jnp.iinfo(jnp.int32).min


def _worker_id():
  return lax.axis_index("s") * NCORES + lax.axis_index("c")


def _key(b):
  return b ^ ((b >> 31) & jnp.int32(0x7FFFFFFF))


def _topk_mask_bits(x_bits):
  mesh = plsc.VectorSubcoreMesh(
      core_axis_name="c", subcore_axis_name="s",
      num_cores=NCORES, num_subcores=NSUB)

  @functools.partial(
      pl.kernel,
      out_type=jax.ShapeDtypeStruct((ROWS, COLS), jnp.int32),
      mesh=mesh,
      compiler_params=pltpu.CompilerParams(needs_layout_passes=False),
      scratch_types=[
          pltpu.VMEM((ROWS_PER_W * COLS,), jnp.int32),
          pltpu.VMEM((NBKT,), jnp.int32),
          pltpu.VMEM((CAND_CAP,), jnp.int32),
          pltpu.SemaphoreType.DMA,
          pltpu.SemaphoreType.DMA,
          pltpu.SemaphoreType.DMA,
          pltpu.SemaphoreType.DMA,
      ],
  )
  def kern(x_hbm, out_hbm, buf_v, hist_v, cand_v, s_in0, s_in1, s_o0, s_o1):
    wid = _worker_id()
    iota = lax.iota(jnp.int32, LANES)
    ones = jnp.ones((LANES,), jnp.int32)
    zeros = jnp.zeros((LANES,), jnp.int32)
    pair = wid * ROWS_PER_W

    cp_in = [
        pltpu.async_copy(x_hbm.at[pair + r],
                         buf_v.at[pl.ds(r * COLS, COLS)],
                         (s_in0, s_in1)[r])
        for r in range(ROWS_PER_W)
    ]
    cp_out = []

    for r_local in range(ROWS_PER_W):
      base = r_local * COLS
      cp_in[r_local].wait()

      @plsc.parallel_loop(0, NBKT, step=LANES, unroll=8)
      def _(i):
        hist_v[pl.ds(i, LANES)] = zeros

      @plsc.parallel_loop(base, base + COLS, step=LANES, unroll=8)
      def _(i):
        b = buf_v[pl.ds(i, LANES)]
        ku = b ^ ((b >> 31) | jnp.int32(-0x80000000))
        bkt = lax.shift_right_logical(ku, BKT_SHIFT)
        plsc.addupdate_scatter(hist_v, [bkt], ones)

      def blk_sum(blk):
        a = hist_v[pl.ds(blk * 64, LANES)]
        for q in range(1, 4):
          a = a + hist_v[pl.ds(blk * 64 + q * LANES, LANES)]
        return jnp.sum(a)

      def scan_cond(c):
        _, above, s = c
        return above + s < KTOP

      def blk_step(c):
        blk, above, s = c
        return blk - 1, above + s, blk_sum(blk - 1)

      top_b = jnp.int32(NBKT // 64 - 1)
      b_star, above_blk, _ = lax.while_loop(
          scan_cond, blk_step, (top_b, jnp.int32(0), blk_sum(top_b)))

      def grp_step(c):
        g, above, s = c
        return g - 1, above + s, jnp.sum(hist_v[pl.ds((g - 1) * LANES, LANES)])

      top_g = b_star * 4 + 3
      g_star, above_g, _ = lax.while_loop(
          scan_cond, grp_step,
          (top_g, above_blk, jnp.sum(hist_v[pl.ds(top_g * LANES, LANES)])))
      totals = hist_v[pl.ds(g_star * LANES, LANES)]
      rev = lax.rev(totals, (0,))
      cum = above_g + plsc.cumsum(rev)
      below = jnp.where(cum < KTOP, 1, 0)
      lstar = jnp.sum(below)
      excl = jnp.sum(jnp.where(cum < KTOP, rev, 0))
      bkt = g_star * LANES + (jnp.int32(LANES - 1) - lstar)
      above_b = above_g + excl

      kprime = jnp.int32(KTOP) - above_b
      bkt_rel = bkt - jnp.int32(NBKT // 2)
      bmin = bkt_rel << BKT_SHIFT

      sentinel = jnp.full((LANES,), I32MIN, jnp.int32)

      @plsc.parallel_loop(0, CAND_FIX, step=LANES, unroll=8)
      def _(i):
        cand_v[pl.ds(i, LANES)] = sentinel

      def collect_raw(_):
        @plsc.parallel_loop(base, base + COLS, step=LANES, unroll=4,
                            carry=jnp.int32(0))
        def c_out(i, c):
          b = buf_v[pl.ds(i, LANES)]
          m = (b >> BKT_SHIFT) == bkt_rel
          plsc.store_compressed(cand_v.at[pl.ds(c, LANES)], b, mask=m)
          return c + jnp.sum(jnp.where(m, 1, 0))

        return c_out

      def collect_keyed(_):
        @plsc.parallel_loop(base, base + COLS, step=LANES, unroll=4,
                            carry=jnp.int32(0))
        def c_out(i, c):
          ks = _key(buf_v[pl.ds(i, LANES)])
          m = (ks >> BKT_SHIFT) == bkt_rel
          plsc.store_compressed(cand_v.at[pl.ds(c, LANES)], ks, mask=m)
          return c + jnp.sum(jnp.where(m, 1, 0))

        return c_out

      cnt = lax.cond(bkt_rel >= 0, collect_raw, collect_keyed, None)
      cand_v[pl.ds(cnt, LANES)] = sentinel

      def search_static(_):
        prefix = bmin
        for bit in range(BKT_SHIFT - 1, -1, -1):
          mid = prefix | (jnp.int32(1) << bit)

          @plsc.parallel_loop(0, CAND_FIX, step=LANES, unroll=8, carry=zeros)
          def acc(i, a, mid=mid):
            kv = cand_v[pl.ds(i, LANES)]
            return a + jnp.where(kv >= mid, 1, 0)

          prefix = jnp.where(jnp.sum(acc) >= kprime, mid, prefix)
        thr = prefix

        @plsc.parallel_loop(0, CAND_FIX, step=LANES, unroll=8,
                            carry=(zeros, zeros))
        def acc2(i, a):
          kv = cand_v[pl.ds(i, LANES)]
          return (a[0] + jnp.where(kv > thr, 1, 0),
                  a[1] + jnp.where(kv == thr, 1, 0))

        return thr, jnp.sum(acc2[0]), jnp.sum(acc2[1])

      def search_dynamic(_):
        ngrp = (cnt + LANES - 1) // LANES
        prefix = bmin
        for bit in range(BKT_SHIFT - 1, -1, -1):
          mid = prefix | (jnp.int32(1) << bit)

          def count_body(i, acc, mid=mid):
            kv = cand_v[pl.ds(i * LANES, LANES)]
            return acc + jnp.where(kv >= mid, 1, 0)

          cge = jnp.sum(lax.fori_loop(0, ngrp, count_body, zeros))
          prefix = jnp.where(cge >= kprime, mid, prefix)
        thr = prefix

        def count2_body(i, acc):
          kv = cand_v[pl.ds(i * LANES, LANES)]
          return (acc[0] + jnp.where(kv > thr, 1, 0),
                  acc[1] + jnp.where(kv == thr, 1, 0))

        a_gt, a_eq = lax.fori_loop(0, ngrp, count2_body, (zeros, zeros))
        return thr, jnp.sum(a_gt), jnp.sum(a_eq)

      thr, c_gt, c_eq = lax.cond(
          cnt <= CAND_FIX - LANES, search_static, search_dynamic, None)
      t_rem = kprime - c_gt

      def find_cut(_):
        def fb(i, carry):
          cnt_eq, cut, done = carry
          ks = _key(buf_v[pl.ds(base + i * LANES, LANES)])
          eq = ks == thr
          eqi = jnp.where(eq, 1, 0)
          cs = plsc.cumsum(eqi)
          tgt = t_rem - cnt_eq
          sel = eq & (cs == tgt)
          gcount = jnp.sum(eqi)
          here = (~done) & (gcount >= tgt)
          idx_in = jnp.sum(jnp.where(sel, iota, 0))
          cut = jnp.where(here, i * LANES + idx_in, cut)
          return cnt_eq + gcount, cut, done | here

        _, cut, _ = lax.fori_loop(
            0, GROUPS, fb, (jnp.int32(0), jnp.int32(COLS - 1), False))
        return cut

      cut = lax.cond(c_eq > t_rem, find_cut,
                     lambda _: jnp.int32(COLS - 1), None)

      fast_ok = (thr > 0) & (cut == jnp.int32(COLS - 1))

      @pl.when(fast_ok)
      def _():
        @plsc.parallel_loop(base, base + COLS, step=LANES, unroll=8)
        def _(i):
          b = buf_v[pl.ds(i, LANES)]
          buf_v[pl.ds(i, LANES)] = jnp.where(b >= thr, b, jnp.int32(0))

      @pl.when(jnp.logical_not(fast_ok))
      def _():
        @plsc.parallel_loop(base, base + COLS, step=LANES, unroll=8)
        def _(i):
          b = buf_v[pl.ds(i, LANES)]
          ks = _key(b)
          idxv = iota + (i - base)
          keep = (ks > thr) | ((ks == thr) & (idxv <= cut))
          buf_v[pl.ds(i, LANES)] = jnp.where(keep, b, jnp.int32(0))

      cp_out.append(
          pltpu.async_copy(buf_v.at[pl.ds(base, COLS)],
                           out_hbm.at[pair + r_local],
                           (s_o0, s_o1)[r_local]))

    for cp in cp_out:
      cp.wait()

  return kern(x_bits)


def kernel(input):
  bits = lax.bitcast_convert_type(input, jnp.int32)
  return lax.bitcast_convert_type(_topk_mask_bits(bits), jnp.float32)
